# Initial kernel scaffold; baseline (speedup 1.0000x reference)
import jax
import jax.numpy as jnp
from jax.experimental import pallas as pl


def kernel(x, weight, edge_index):
    raise NotImplementedError("write your pallas kernel here")


# speedup vs baseline: 12.7845x; 12.7845x over previous
import functools

import jax
import jax.numpy as jnp
from jax import lax
from jax.experimental import pallas as pl
from jax.experimental.pallas import tpu as pltpu


def _round_up(a, b):
    return ((a + b - 1) // b) * b


def _xw_kernel(x_ref, w_ref, out_ref):
    out_ref[...] = jnp.dot(
        x_ref[...], w_ref[...], preferred_element_type=jnp.float32
    ).astype(out_ref.dtype)


def _gather_kernel(ec_ref, el_ref, xw_ref, gd_ref, gs_ref):
    TE = gd_ref.shape[0]
    Np = xw_ref.shape[0]

    a_t = ec_ref[:, 0:1]
    b_t = ec_ref[:, 1:2]
    key_t = ec_ref[:, 2:3]
    key_all = el_ref[2:3, :]

    count = jnp.sum((key_t == key_all).astype(jnp.float32), axis=1,
                    keepdims=True)
    mult = count * (1.0 + (a_t == b_t).astype(jnp.float32))
    w = 1.0 / mult

    col_ids = lax.broadcasted_iota(jnp.int32, (TE, Np), 1)
    oh_d = (b_t == col_ids).astype(xw_ref.dtype)
    oh_s = (a_t == col_ids).astype(xw_ref.dtype)

    gd = jnp.dot(oh_d, xw_ref[...], preferred_element_type=jnp.float32)
    gs = jnp.dot(oh_s, xw_ref[...], preferred_element_type=jnp.float32)
    gd_ref[...] = (w * gd).astype(gd_ref.dtype)
    gs_ref[...] = (w * gs).astype(gs_ref.dtype)


def _scatter_kernel(el_ref, gd_ref, gs_ref, out_ref):
    TM = out_ref.shape[0]
    Ep = gd_ref.shape[0]
    i = pl.program_id(0)

    row_ids = lax.broadcasted_iota(jnp.int32, (TM, Ep), 0) + i * TM
    s_oh = (row_ids == el_ref[0:1, :]).astype(gd_ref.dtype)
    d_oh = (row_ids == el_ref[1:2, :]).astype(gs_ref.dtype)

    acc = jnp.dot(s_oh, gd_ref[...], preferred_element_type=jnp.float32)
    acc += jnp.dot(d_oh, gs_ref[...], preferred_element_type=jnp.float32)
    out_ref[...] = jnp.maximum(acc, 0.0).astype(out_ref.dtype)


@functools.partial(jax.jit, static_argnums=(3,))
def _graph_conv(x, weight, edge_index, num_nodes):
    N = num_nodes
    D_in = x.shape[1]
    D_out = weight.shape[1]
    E = edge_index.shape[1]

    TM1 = 512
    TE = 256
    TM = 256
    Np = _round_up(max(N, 1), 256)
    Dk = _round_up(D_in, 128)
    Do = _round_up(D_out, 128)
    Ep = _round_up(max(E, 1), TE)

    bf16 = jnp.bfloat16
    f32 = jnp.float32
    x_p = jnp.zeros((Np, Dk), bf16).at[:N, :D_in].set(x.astype(bf16))
    w_p = jnp.zeros((Dk, Do), bf16).at[:D_in, :D_out].set(weight.astype(bf16))

    e = edge_index.astype(jnp.int32)
    a = jnp.full((Ep,), -1, jnp.int32).at[:E].set(e[0])
    b = jnp.full((Ep,), -1, jnp.int32).at[:E].set(e[1])
    key = jnp.minimum(a, b) * N + jnp.maximum(a, b)
    e_lane = jnp.zeros((8, Ep), jnp.int32)
    e_lane = e_lane.at[0].set(a).at[1].set(b).at[2].set(key)
    e_col = e_lane.T

    xw = pl.pallas_call(
        _xw_kernel,
        out_shape=jax.ShapeDtypeStruct((Np, Do), bf16),
        grid=(Np // TM1,),
        in_specs=[
            pl.BlockSpec((TM1, Dk), lambda i: (i, 0)),
            pl.BlockSpec((Dk, Do), lambda i: (0, 0)),
        ],
        out_specs=pl.BlockSpec((TM1, Do), lambda i: (i, 0)),
        compiler_params=pltpu.CompilerParams(
            dimension_semantics=("parallel",),
            vmem_limit_bytes=48 << 20,
        ),
    )(x_p, w_p)

    gd, gs = pl.pallas_call(
        _gather_kernel,
        out_shape=[
            jax.ShapeDtypeStruct((Ep, Do), bf16),
            jax.ShapeDtypeStruct((Ep, Do), bf16),
        ],
        grid=(Ep // TE,),
        in_specs=[
            pl.BlockSpec((TE, 8), lambda t: (t, 0)),
            pl.BlockSpec((8, Ep), lambda t: (0, 0)),
            pl.BlockSpec((Np, Do), lambda t: (0, 0)),
        ],
        out_specs=[
            pl.BlockSpec((TE, Do), lambda t: (t, 0)),
            pl.BlockSpec((TE, Do), lambda t: (t, 0)),
        ],
        compiler_params=pltpu.CompilerParams(
            dimension_semantics=("parallel",),
            vmem_limit_bytes=56 << 20,
        ),
    )(e_col, e_lane, xw)

    out_p = pl.pallas_call(
        _scatter_kernel,
        out_shape=jax.ShapeDtypeStruct((Np, Do), f32),
        grid=(Np // TM,),
        in_specs=[
            pl.BlockSpec((8, Ep), lambda i: (0, 0)),
            pl.BlockSpec((Ep, Do), lambda i: (0, 0)),
            pl.BlockSpec((Ep, Do), lambda i: (0, 0)),
        ],
        out_specs=pl.BlockSpec((TM, Do), lambda i: (i, 0)),
        compiler_params=pltpu.CompilerParams(
            dimension_semantics=("parallel",),
            vmem_limit_bytes=56 << 20,
        ),
    )(e_lane, gd, gs)

    return out_p[:N, :D_out]


def kernel(x, weight, edge_index):
    return _graph_conv(x, weight, edge_index, 4096)


# speedup vs baseline: 13.4772x; 1.0542x over previous
import functools

import jax
import jax.numpy as jnp
from jax import lax
from jax.experimental import pallas as pl
from jax.experimental.pallas import tpu as pltpu


def _round_up(a, b):
    return ((a + b - 1) // b) * b


def _xw_kernel(x_ref, w_ref, out_ref):
    out_ref[...] = jnp.dot(
        x_ref[...].astype(jnp.bfloat16),
        w_ref[...].astype(jnp.bfloat16),
        preferred_element_type=jnp.float32,
    ).astype(out_ref.dtype)


def _gather_kernel(ec_ref, el_ref, xw_ref, gd_ref, gs_ref):
    TE = gd_ref.shape[0]
    Np = xw_ref.shape[0]

    a_t = ec_ref[:, 0:1]
    b_t = ec_ref[:, 1:2]
    key_t = ec_ref[:, 2:3]
    key_all = el_ref[2:3, :]

    count = jnp.sum((key_t == key_all).astype(jnp.float32), axis=1,
                    keepdims=True)
    mult = count * (1.0 + (a_t == b_t).astype(jnp.float32))
    w = 1.0 / mult

    col_ids = lax.broadcasted_iota(jnp.int32, (TE, Np), 1)
    oh_d = (b_t == col_ids).astype(xw_ref.dtype)
    oh_s = (a_t == col_ids).astype(xw_ref.dtype)

    gd = jnp.dot(oh_d, xw_ref[...], preferred_element_type=jnp.float32)
    gs = jnp.dot(oh_s, xw_ref[...], preferred_element_type=jnp.float32)
    gd_ref[...] = (w * gd).astype(gd_ref.dtype)
    gs_ref[...] = (w * gs).astype(gs_ref.dtype)


def _scatter_kernel(el_ref, gd_ref, gs_ref, out_ref):
    TM = out_ref.shape[0]
    Ep = gd_ref.shape[0]
    i = pl.program_id(0)

    row_ids = lax.broadcasted_iota(jnp.int32, (TM, Ep), 0) + i * TM
    s_oh = (row_ids == el_ref[0:1, :]).astype(gd_ref.dtype)
    d_oh = (row_ids == el_ref[1:2, :]).astype(gs_ref.dtype)

    acc = jnp.dot(s_oh, gd_ref[...], preferred_element_type=jnp.float32)
    acc += jnp.dot(d_oh, gs_ref[...], preferred_element_type=jnp.float32)
    out_ref[...] = jnp.maximum(acc, 0.0).astype(out_ref.dtype)


@functools.partial(jax.jit, static_argnums=(3,))
def _graph_conv(x, weight, edge_index, num_nodes):
    N = num_nodes
    D_in = x.shape[1]
    D_out = weight.shape[1]
    E = edge_index.shape[1]

    TM1 = 1024
    TE = 256
    TM = 256
    Np = _round_up(max(N, 1), 256)
    Dk = _round_up(D_in, 128)
    Do = _round_up(D_out, 128)
    Ep = _round_up(max(E, 1), TE)
    TM1 = min(TM1, Np)

    bf16 = jnp.bfloat16
    f32 = jnp.float32
    if (N, D_in) == (Np, Dk):
        x_p = x
    else:
        x_p = jnp.zeros((Np, Dk), x.dtype).at[:N, :D_in].set(x)
    if (D_in, D_out) == (Dk, Do):
        w_p = weight
    else:
        w_p = jnp.zeros((Dk, Do), weight.dtype).at[:D_in, :D_out].set(weight)

    e = edge_index.astype(jnp.int32)
    a = jnp.full((Ep,), -1, jnp.int32).at[:E].set(e[0])
    b = jnp.full((Ep,), -1, jnp.int32).at[:E].set(e[1])
    key = jnp.minimum(a, b) * N + jnp.maximum(a, b)
    e_lane = jnp.zeros((8, Ep), jnp.int32)
    e_lane = e_lane.at[0].set(a).at[1].set(b).at[2].set(key)
    e_col = e_lane.T

    xw = pl.pallas_call(
        _xw_kernel,
        out_shape=jax.ShapeDtypeStruct((Np, Do), bf16),
        grid=(Np // TM1,),
        in_specs=[
            pl.BlockSpec((TM1, Dk), lambda i: (i, 0)),
            pl.BlockSpec((Dk, Do), lambda i: (0, 0)),
        ],
        out_specs=pl.BlockSpec((TM1, Do), lambda i: (i, 0)),
        compiler_params=pltpu.CompilerParams(
            dimension_semantics=("parallel",),
            vmem_limit_bytes=48 << 20,
        ),
    )(x_p, w_p)

    gd, gs = pl.pallas_call(
        _gather_kernel,
        out_shape=[
            jax.ShapeDtypeStruct((Ep, Do), bf16),
            jax.ShapeDtypeStruct((Ep, Do), bf16),
        ],
        grid=(Ep // TE,),
        in_specs=[
            pl.BlockSpec((TE, 8), lambda t: (t, 0)),
            pl.BlockSpec((8, Ep), lambda t: (0, 0)),
            pl.BlockSpec((Np, Do), lambda t: (0, 0)),
        ],
        out_specs=[
            pl.BlockSpec((TE, Do), lambda t: (t, 0)),
            pl.BlockSpec((TE, Do), lambda t: (t, 0)),
        ],
        compiler_params=pltpu.CompilerParams(
            dimension_semantics=("parallel",),
            vmem_limit_bytes=56 << 20,
        ),
    )(e_col, e_lane, xw)

    out_p = pl.pallas_call(
        _scatter_kernel,
        out_shape=jax.ShapeDtypeStruct((Np, Do), f32),
        grid=(Np // TM,),
        in_specs=[
            pl.BlockSpec((8, Ep), lambda i: (0, 0)),
            pl.BlockSpec((Ep, Do), lambda i: (0, 0)),
            pl.BlockSpec((Ep, Do), lambda i: (0, 0)),
        ],
        out_specs=pl.BlockSpec((TM, Do), lambda i: (i, 0)),
        compiler_params=pltpu.CompilerParams(
            dimension_semantics=("parallel",),
            vmem_limit_bytes=56 << 20,
        ),
    )(e_lane, gd, gs)

    return out_p[:N, :D_out]


def kernel(x, weight, edge_index):
    return _graph_conv(x, weight, edge_index, 4096)


# speedup vs baseline: 13.7754x; 1.0221x over previous
import functools

import jax
import jax.numpy as jnp
from jax import lax
from jax.experimental import pallas as pl
from jax.experimental.pallas import tpu as pltpu


def _round_up(a, b):
    return ((a + b - 1) // b) * b


def _xw_kernel(x_ref, w_ref, out_ref):
    out_ref[...] = jnp.dot(
        x_ref[...].astype(jnp.bfloat16),
        w_ref[...].astype(jnp.bfloat16),
        preferred_element_type=jnp.float32,
    ).astype(out_ref.dtype)


def _gather_kernel(ec_ref, el_ref, xw_ref, gd_ref, gs_ref):
    TE = gd_ref.shape[0]
    Np = xw_ref.shape[0]

    a_t = ec_ref[:, 0:1]
    b_t = ec_ref[:, 1:2]
    key_t = ec_ref[:, 2:3]
    key_all = el_ref[2:3, :]

    count = jnp.sum((key_t == key_all).astype(jnp.float32), axis=1,
                    keepdims=True)
    mult = count * (1.0 + (a_t == b_t).astype(jnp.float32))
    w = 1.0 / mult

    col_ids = lax.broadcasted_iota(jnp.int32, (TE, Np), 1)
    oh_d = (b_t == col_ids).astype(xw_ref.dtype)
    oh_s = (a_t == col_ids).astype(xw_ref.dtype)

    gd = jnp.dot(oh_d, xw_ref[...], preferred_element_type=jnp.float32)
    gs = jnp.dot(oh_s, xw_ref[...], preferred_element_type=jnp.float32)
    gd_ref[...] = (w * gd).astype(gd_ref.dtype)
    gs_ref[...] = (w * gs).astype(gs_ref.dtype)


def _scatter_kernel(el_ref, gd_ref, gs_ref, out_ref):
    TM = out_ref.shape[0]
    Ep = gd_ref.shape[0]
    i = pl.program_id(0)

    row_ids = lax.broadcasted_iota(jnp.int32, (TM, Ep), 0) + i * TM
    s_oh = (row_ids == el_ref[0:1, :]).astype(gd_ref.dtype)
    d_oh = (row_ids == el_ref[1:2, :]).astype(gs_ref.dtype)

    acc = jnp.dot(s_oh, gd_ref[...], preferred_element_type=jnp.float32)
    acc += jnp.dot(d_oh, gs_ref[...], preferred_element_type=jnp.float32)
    out_ref[...] = jnp.maximum(acc, 0.0).astype(out_ref.dtype)


@functools.partial(jax.jit, static_argnums=(3,))
def _graph_conv(x, weight, edge_index, num_nodes):
    N = num_nodes
    D_in = x.shape[1]
    D_out = weight.shape[1]
    E = edge_index.shape[1]

    TM1 = 1024
    TE = 512
    TM = 512
    Np = _round_up(max(N, 1), 256)
    Dk = _round_up(D_in, 128)
    Do = _round_up(D_out, 128)
    Ep = _round_up(max(E, 1), TE)
    TM1 = min(TM1, Np)

    bf16 = jnp.bfloat16
    f32 = jnp.float32
    if (N, D_in) == (Np, Dk):
        x_p = x
    else:
        x_p = jnp.zeros((Np, Dk), x.dtype).at[:N, :D_in].set(x)
    if (D_in, D_out) == (Dk, Do):
        w_p = weight
    else:
        w_p = jnp.zeros((Dk, Do), weight.dtype).at[:D_in, :D_out].set(weight)

    e = edge_index.astype(jnp.int32)
    a = jnp.full((Ep,), -1, jnp.int32).at[:E].set(e[0])
    b = jnp.full((Ep,), -1, jnp.int32).at[:E].set(e[1])
    key = jnp.minimum(a, b) * N + jnp.maximum(a, b)
    e_lane = jnp.zeros((8, Ep), jnp.int32)
    e_lane = e_lane.at[0].set(a).at[1].set(b).at[2].set(key)
    e_col = e_lane.T

    xw = pl.pallas_call(
        _xw_kernel,
        out_shape=jax.ShapeDtypeStruct((Np, Do), bf16),
        grid=(Np // TM1,),
        in_specs=[
            pl.BlockSpec((TM1, Dk), lambda i: (i, 0)),
            pl.BlockSpec((Dk, Do), lambda i: (0, 0)),
        ],
        out_specs=pl.BlockSpec((TM1, Do), lambda i: (i, 0)),
        compiler_params=pltpu.CompilerParams(
            dimension_semantics=("parallel",),
            vmem_limit_bytes=48 << 20,
        ),
    )(x_p, w_p)

    gd, gs = pl.pallas_call(
        _gather_kernel,
        out_shape=[
            jax.ShapeDtypeStruct((Ep, Do), bf16),
            jax.ShapeDtypeStruct((Ep, Do), bf16),
        ],
        grid=(Ep // TE,),
        in_specs=[
            pl.BlockSpec((TE, 8), lambda t: (t, 0)),
            pl.BlockSpec((8, Ep), lambda t: (0, 0)),
            pl.BlockSpec((Np, Do), lambda t: (0, 0)),
        ],
        out_specs=[
            pl.BlockSpec((TE, Do), lambda t: (t, 0)),
            pl.BlockSpec((TE, Do), lambda t: (t, 0)),
        ],
        compiler_params=pltpu.CompilerParams(
            dimension_semantics=("parallel",),
            vmem_limit_bytes=56 << 20,
        ),
    )(e_col, e_lane, xw)

    out_p = pl.pallas_call(
        _scatter_kernel,
        out_shape=jax.ShapeDtypeStruct((Np, Do), f32),
        grid=(Np // TM,),
        in_specs=[
            pl.BlockSpec((8, Ep), lambda i: (0, 0)),
            pl.BlockSpec((Ep, Do), lambda i: (0, 0)),
            pl.BlockSpec((Ep, Do), lambda i: (0, 0)),
        ],
        out_specs=pl.BlockSpec((TM, Do), lambda i: (i, 0)),
        compiler_params=pltpu.CompilerParams(
            dimension_semantics=("parallel",),
            vmem_limit_bytes=56 << 20,
        ),
    )(e_lane, gd, gs)

    return out_p[:N, :D_out]


def kernel(x, weight, edge_index):
    return _graph_conv(x, weight, edge_index, 4096)


# speedup vs baseline: 13.9355x; 1.0116x over previous
import functools

import jax
import jax.numpy as jnp
from jax import lax
from jax.experimental import pallas as pl
from jax.experimental.pallas import tpu as pltpu


def _round_up(a, b):
    return ((a + b - 1) // b) * b


def _xw_kernel(x_ref, w_ref, out_ref):
    out_ref[...] = jnp.dot(
        x_ref[...].astype(jnp.bfloat16),
        w_ref[...].astype(jnp.bfloat16),
        preferred_element_type=jnp.float32,
    ).astype(out_ref.dtype)


def _gather_kernel(ec_ref, el_ref, xw_ref, gd_ref, gs_ref):
    TE = gd_ref.shape[0]
    Np = xw_ref.shape[0]

    a_t = ec_ref[:, 0:1]
    b_t = ec_ref[:, 1:2]
    key_t = ec_ref[:, 2:3]
    key_all = el_ref[2:3, :]

    count = jnp.sum((key_t == key_all).astype(jnp.float32), axis=1,
                    keepdims=True)
    mult = count * (1.0 + (a_t == b_t).astype(jnp.float32))
    w = 1.0 / mult

    col_ids = lax.broadcasted_iota(jnp.int32, (TE, Np), 1)
    oh_d = (b_t == col_ids).astype(xw_ref.dtype)
    oh_s = (a_t == col_ids).astype(xw_ref.dtype)

    gd = jnp.dot(oh_d, xw_ref[...], preferred_element_type=jnp.float32)
    gs = jnp.dot(oh_s, xw_ref[...], preferred_element_type=jnp.float32)
    gd_ref[...] = (w * gd).astype(gd_ref.dtype)
    gs_ref[...] = (w * gs).astype(gs_ref.dtype)


def _scatter_kernel(el_ref, gd_ref, gs_ref, out_ref):
    TM = out_ref.shape[0]
    Ep = gd_ref.shape[0]
    i = pl.program_id(0)

    row_ids = lax.broadcasted_iota(jnp.int32, (TM, Ep), 0) + i * TM
    s_oh = (row_ids == el_ref[0:1, :]).astype(gd_ref.dtype)
    d_oh = (row_ids == el_ref[1:2, :]).astype(gs_ref.dtype)

    acc = jnp.dot(s_oh, gd_ref[...], preferred_element_type=jnp.float32)
    acc += jnp.dot(d_oh, gs_ref[...], preferred_element_type=jnp.float32)
    out_ref[...] = jnp.maximum(acc, 0.0).astype(out_ref.dtype)


@functools.partial(jax.jit, static_argnums=(3,))
def _graph_conv(x, weight, edge_index, num_nodes):
    N = num_nodes
    D_in = x.shape[1]
    D_out = weight.shape[1]
    E = edge_index.shape[1]

    TM1 = 1024
    TE = 1024
    TM = 1024
    Np = _round_up(max(N, 1), 256)
    Dk = _round_up(D_in, 128)
    Do = _round_up(D_out, 128)
    Ep = _round_up(max(E, 1), TE)
    TM1 = min(TM1, Np)

    bf16 = jnp.bfloat16
    f32 = jnp.float32
    if (N, D_in) == (Np, Dk):
        x_p = x
    else:
        x_p = jnp.zeros((Np, Dk), x.dtype).at[:N, :D_in].set(x)
    if (D_in, D_out) == (Dk, Do):
        w_p = weight
    else:
        w_p = jnp.zeros((Dk, Do), weight.dtype).at[:D_in, :D_out].set(weight)

    e = edge_index.astype(jnp.int32)
    a = jnp.full((Ep,), -1, jnp.int32).at[:E].set(e[0])
    b = jnp.full((Ep,), -1, jnp.int32).at[:E].set(e[1])
    key = jnp.minimum(a, b) * N + jnp.maximum(a, b)
    e_lane = jnp.zeros((8, Ep), jnp.int32)
    e_lane = e_lane.at[0].set(a).at[1].set(b).at[2].set(key)
    e_col = e_lane.T

    xw = pl.pallas_call(
        _xw_kernel,
        out_shape=jax.ShapeDtypeStruct((Np, Do), bf16),
        grid=(Np // TM1,),
        in_specs=[
            pl.BlockSpec((TM1, Dk), lambda i: (i, 0)),
            pl.BlockSpec((Dk, Do), lambda i: (0, 0)),
        ],
        out_specs=pl.BlockSpec((TM1, Do), lambda i: (i, 0)),
        compiler_params=pltpu.CompilerParams(
            dimension_semantics=("parallel",),
            vmem_limit_bytes=48 << 20,
        ),
    )(x_p, w_p)

    gd, gs = pl.pallas_call(
        _gather_kernel,
        out_shape=[
            jax.ShapeDtypeStruct((Ep, Do), bf16),
            jax.ShapeDtypeStruct((Ep, Do), bf16),
        ],
        grid=(Ep // TE,),
        in_specs=[
            pl.BlockSpec((TE, 8), lambda t: (t, 0)),
            pl.BlockSpec((8, Ep), lambda t: (0, 0)),
            pl.BlockSpec((Np, Do), lambda t: (0, 0)),
        ],
        out_specs=[
            pl.BlockSpec((TE, Do), lambda t: (t, 0)),
            pl.BlockSpec((TE, Do), lambda t: (t, 0)),
        ],
        compiler_params=pltpu.CompilerParams(
            dimension_semantics=("parallel",),
            vmem_limit_bytes=56 << 20,
        ),
    )(e_col, e_lane, xw)

    out_p = pl.pallas_call(
        _scatter_kernel,
        out_shape=jax.ShapeDtypeStruct((Np, Do), f32),
        grid=(Np // TM,),
        in_specs=[
            pl.BlockSpec((8, Ep), lambda i: (0, 0)),
            pl.BlockSpec((Ep, Do), lambda i: (0, 0)),
            pl.BlockSpec((Ep, Do), lambda i: (0, 0)),
        ],
        out_specs=pl.BlockSpec((TM, Do), lambda i: (i, 0)),
        compiler_params=pltpu.CompilerParams(
            dimension_semantics=("parallel",),
            vmem_limit_bytes=56 << 20,
        ),
    )(e_lane, gd, gs)

    return out_p[:N, :D_out]


def kernel(x, weight, edge_index):
    return _graph_conv(x, weight, edge_index, 4096)
